# baseline (speedup 1.0000x reference)
import jax
import jax.numpy as jnp
from jax import lax
from jax.experimental import pallas as pl
from jax.experimental.pallas import tpu as pltpu

NUM_FEATURES = 100
NUM_FIELDS = 26
EMBED = 16
FLAT = NUM_FEATURES * EMBED
BLK = 1024


def _fm_body(x_ref, w_ref, V_ref, fi_ref, yfm_ref, out_ref,
             r_ref, er_ref, a_ref, q_ref):
    f32 = jnp.float32
    hi = lax.Precision.HIGHEST

    @pl.when(pl.program_id(0) == 0)
    def _prep():
        fi = fi_ref[:]
        onehot = (fi == lax.broadcasted_iota(
            jnp.int32, (NUM_FEATURES, NUM_FIELDS), 1)).astype(f32)
        emb = jnp.dot(onehot, V_ref[:], precision=hi,
                      preferred_element_type=f32)
        t_row = lax.broadcasted_iota(jnp.int32, (EMBED, FLAT), 0)
        t_col = lax.broadcasted_iota(jnp.int32, (EMBED, FLAT), 1)
        tmat = (t_col % EMBED == t_row).astype(f32)
        m_row = lax.broadcasted_iota(jnp.int32, (NUM_FEATURES, FLAT), 0)
        m_col = lax.broadcasted_iota(jnp.int32, (NUM_FEATURES, FLAT), 1)
        mask = (m_col // EMBED == m_row).astype(f32)
        mmat = jnp.dot(emb, tmat, precision=hi,
                       preferred_element_type=f32) * mask
        r_ref[:] = mask.astype(jnp.bfloat16)
        er_ref[:] = jnp.sum(mmat, axis=0, keepdims=True)
        rowsum = jnp.sum(emb, axis=1, keepdims=True)
        a_ref[:] = jnp.concatenate([w_ref[:], rowsum], axis=1)
        q_ref[:] = jnp.sum(emb * emb, axis=1, keepdims=True)

    xb = x_ref[:]
    xhi = xb.astype(jnp.bfloat16)
    xlo = (xb - xhi.astype(f32)).astype(jnp.bfloat16)
    rmat = r_ref[:]
    rep = (jnp.dot(xhi, rmat, preferred_element_type=f32) +
           jnp.dot(xlo, rmat, preferred_element_type=f32))
    out_ref[:] = rep * er_ref[:]

    p = jnp.dot(xb, a_ref[:], precision=hi,
                preferred_element_type=f32)
    sq = jnp.dot(xb * xb, q_ref[:], precision=hi,
                 preferred_element_type=f32)
    inter = 0.5 * (p[:, 1:2] * p[:, 1:2] - sq)
    yfm_ref[:] = jnp.concatenate([p[:, 0:1], inter], axis=1)


def kernel(x, w, V, field_index):
    batch = x.shape[0]
    w2 = w.reshape(NUM_FEATURES, 1)
    fi2 = field_index.reshape(NUM_FEATURES, 1)
    grid = batch // BLK
    yfm, flat = pl.pallas_call(
        _fm_body,
        grid=(grid,),
        in_specs=[
            pl.BlockSpec((BLK, NUM_FEATURES), lambda i: (i, 0)),
            pl.BlockSpec((NUM_FEATURES, 1), lambda i: (0, 0)),
            pl.BlockSpec((NUM_FIELDS, EMBED), lambda i: (0, 0)),
            pl.BlockSpec((NUM_FEATURES, 1), lambda i: (0, 0)),
        ],
        out_specs=[
            pl.BlockSpec((BLK, 2), lambda i: (i, 0)),
            pl.BlockSpec((BLK, FLAT), lambda i: (i, 0)),
        ],
        out_shape=[
            jax.ShapeDtypeStruct((batch, 2), jnp.float32),
            jax.ShapeDtypeStruct((batch, FLAT), jnp.float32),
        ],
        scratch_shapes=[
            pltpu.VMEM((NUM_FEATURES, FLAT), jnp.bfloat16),
            pltpu.VMEM((1, FLAT), jnp.float32),
            pltpu.VMEM((NUM_FEATURES, 2), jnp.float32),
            pltpu.VMEM((NUM_FEATURES, 1), jnp.float32),
        ],
        compiler_params=pltpu.CompilerParams(
            dimension_semantics=("arbitrary",)),
    )(x, w2, V, fi2)
    return (yfm, flat.reshape(batch, NUM_FEATURES, EMBED))


# speedup vs baseline: 1.0461x; 1.0461x over previous
import jax
import jax.numpy as jnp
from jax import lax
from jax.experimental import pallas as pl
from jax.experimental.pallas import tpu as pltpu

NUM_FEATURES = 100
NUM_FIELDS = 26
EMBED = 16
FLAT = NUM_FEATURES * EMBED
BLK = 1024


def _fm_body(x_ref, w_ref, V_ref, fi_ref, yfm_ref, out_ref,
             er_ref, a_ref, q_ref):
    f32 = jnp.float32
    hi = lax.Precision.HIGHEST

    @pl.when(pl.program_id(0) == 0)
    def _prep():
        fi = fi_ref[:]
        onehot = (fi == lax.broadcasted_iota(
            jnp.int32, (NUM_FEATURES, NUM_FIELDS), 1)).astype(f32)
        emb = jnp.dot(onehot, V_ref[:], precision=hi,
                      preferred_element_type=f32)
        t_row = lax.broadcasted_iota(jnp.int32, (EMBED, FLAT), 0)
        t_col = lax.broadcasted_iota(jnp.int32, (EMBED, FLAT), 1)
        tmat = (t_col % EMBED == t_row).astype(f32)
        m_row = lax.broadcasted_iota(jnp.int32, (NUM_FEATURES, FLAT), 0)
        m_col = lax.broadcasted_iota(jnp.int32, (NUM_FEATURES, FLAT), 1)
        mask = (m_col // EMBED == m_row).astype(f32)
        mmat = jnp.dot(emb, tmat, precision=hi,
                       preferred_element_type=f32) * mask
        er_ref[:] = jnp.sum(mmat, axis=0, keepdims=True)
        rowsum = jnp.sum(emb, axis=1, keepdims=True)
        a_ref[:] = jnp.concatenate([w_ref[:], rowsum], axis=1)
        q_ref[:] = jnp.sum(emb * emb, axis=1, keepdims=True)

    xb = x_ref[:]
    for k in range(FLAT // 128):
        idx = (lax.broadcasted_iota(jnp.int32, (1, 128), 1) + 128 * k) // EMBED
        rep_k = jnp.take_along_axis(
            xb, jnp.broadcast_to(idx, (BLK, 128)), axis=1)
        out_ref[:, 128 * k:128 * (k + 1)] = rep_k * er_ref[:, 128 * k:128 * (k + 1)]
    rem = FLAT - 128 * (FLAT // 128)
    if rem:
        idx = (lax.broadcasted_iota(jnp.int32, (1, rem), 1)
               + 128 * (FLAT // 128)) // EMBED
        rep_k = jnp.take_along_axis(
            xb, jnp.broadcast_to(idx, (BLK, rem)), axis=1)
        out_ref[:, 128 * (FLAT // 128):] = (
            rep_k * er_ref[:, 128 * (FLAT // 128):])

    p = jnp.dot(xb, a_ref[:], precision=hi,
                preferred_element_type=f32)
    sq = jnp.dot(xb * xb, q_ref[:], precision=hi,
                 preferred_element_type=f32)
    inter = 0.5 * (p[:, 1:2] * p[:, 1:2] - sq)
    yfm_ref[:] = jnp.concatenate([p[:, 0:1], inter], axis=1)


def kernel(x, w, V, field_index):
    batch = x.shape[0]
    w2 = w.reshape(NUM_FEATURES, 1)
    fi2 = field_index.reshape(NUM_FEATURES, 1)
    grid = batch // BLK
    yfm, flat = pl.pallas_call(
        _fm_body,
        grid=(grid,),
        in_specs=[
            pl.BlockSpec((BLK, NUM_FEATURES), lambda i: (i, 0)),
            pl.BlockSpec((NUM_FEATURES, 1), lambda i: (0, 0)),
            pl.BlockSpec((NUM_FIELDS, EMBED), lambda i: (0, 0)),
            pl.BlockSpec((NUM_FEATURES, 1), lambda i: (0, 0)),
        ],
        out_specs=[
            pl.BlockSpec((BLK, 2), lambda i: (i, 0)),
            pl.BlockSpec((BLK, FLAT), lambda i: (i, 0)),
        ],
        out_shape=[
            jax.ShapeDtypeStruct((batch, 2), jnp.float32),
            jax.ShapeDtypeStruct((batch, FLAT), jnp.float32),
        ],
        scratch_shapes=[
            pltpu.VMEM((1, FLAT), jnp.float32),
            pltpu.VMEM((NUM_FEATURES, 2), jnp.float32),
            pltpu.VMEM((NUM_FEATURES, 1), jnp.float32),
        ],
        compiler_params=pltpu.CompilerParams(
            dimension_semantics=("arbitrary",)),
    )(x, w2, V, fi2)
    return (yfm, flat.reshape(batch, NUM_FEATURES, EMBED))


# speedup vs baseline: 3.2762x; 3.1320x over previous
import jax
import jax.numpy as jnp
from jax import lax
from jax.experimental import pallas as pl
from jax.experimental.pallas import tpu as pltpu

NUM_FEATURES = 100
NUM_FIELDS = 26
EMBED = 16
CB = 2048


def _fm_body(xt_ref, w_ref, V_ref, fi_ref, yfm_ref, out_ref,
             emb_ref, a_ref, q_ref):
    f32 = jnp.float32
    hi = lax.Precision.HIGHEST

    @pl.when(pl.program_id(0) == 0)
    def _prep():
        fi = fi_ref[:]
        onehot = (fi == lax.broadcasted_iota(
            jnp.int32, (NUM_FEATURES, NUM_FIELDS), 1)).astype(f32)
        emb = jnp.dot(onehot, V_ref[:], precision=hi,
                      preferred_element_type=f32)
        emb_ref[:] = emb
        rowsum = jnp.sum(emb, axis=1, keepdims=True)
        a_ref[:] = jnp.concatenate([w_ref[:], rowsum], axis=1)
        q_ref[:] = jnp.sum(emb * emb, axis=1, keepdims=True)

    xtb = xt_ref[:]
    emb = emb_ref[:]
    for e in range(EMBED):
        out_ref[:, e, :] = xtb * emb[:, e:e + 1]

    p = lax.dot_general(a_ref[:], xtb, (((0,), (0,)), ((), ())),
                        precision=hi, preferred_element_type=f32)
    sq = lax.dot_general(q_ref[:], xtb * xtb, (((0,), (0,)), ((), ())),
                         precision=hi, preferred_element_type=f32)
    inter = 0.5 * (p[1:2] * p[1:2] - sq)
    yfm_ref[:] = jnp.concatenate([p[0:1], inter], axis=0)


def kernel(x, w, V, field_index):
    batch = x.shape[0]
    xt = x.T
    w2 = w.reshape(NUM_FEATURES, 1)
    fi2 = field_index.reshape(NUM_FEATURES, 1)
    grid = batch // CB
    yfm_t, out_p = pl.pallas_call(
        _fm_body,
        grid=(grid,),
        in_specs=[
            pl.BlockSpec((NUM_FEATURES, CB), lambda i: (0, i)),
            pl.BlockSpec((NUM_FEATURES, 1), lambda i: (0, 0)),
            pl.BlockSpec((NUM_FIELDS, EMBED), lambda i: (0, 0)),
            pl.BlockSpec((NUM_FEATURES, 1), lambda i: (0, 0)),
        ],
        out_specs=[
            pl.BlockSpec((2, CB), lambda i: (0, i)),
            pl.BlockSpec((NUM_FEATURES, EMBED, CB), lambda i: (0, 0, i)),
        ],
        out_shape=[
            jax.ShapeDtypeStruct((2, batch), jnp.float32),
            jax.ShapeDtypeStruct((NUM_FEATURES, EMBED, batch), jnp.float32),
        ],
        scratch_shapes=[
            pltpu.VMEM((NUM_FEATURES, EMBED), jnp.float32),
            pltpu.VMEM((NUM_FEATURES, 2), jnp.float32),
            pltpu.VMEM((NUM_FEATURES, 1), jnp.float32),
        ],
        compiler_params=pltpu.CompilerParams(
            dimension_semantics=("arbitrary",)),
    )(xt, w2, V, fi2)
    return (yfm_t.T, jnp.transpose(out_p, (2, 0, 1)))


# speedup vs baseline: 3.4162x; 1.0427x over previous
import jax
import jax.numpy as jnp
from jax import lax
from jax.experimental import pallas as pl
from jax.experimental.pallas import tpu as pltpu

NUM_FEATURES = 100
NUM_FIELDS = 26
EMBED = 16
CB = 1024


def _fm_body(xt_ref, w_ref, V_ref, fi_ref, yfm_ref, out_ref,
             emb_ref, a_ref, q_ref):
    f32 = jnp.float32
    hi = lax.Precision.HIGHEST

    @pl.when(pl.program_id(0) == 0)
    def _prep():
        fi = fi_ref[:]
        onehot = (fi == lax.broadcasted_iota(
            jnp.int32, (NUM_FEATURES, NUM_FIELDS), 1)).astype(f32)
        emb = jnp.dot(onehot, V_ref[:], precision=hi,
                      preferred_element_type=f32)
        emb_ref[:] = emb
        rowsum = jnp.sum(emb, axis=1, keepdims=True)
        a_ref[:] = jnp.concatenate([w_ref[:], rowsum], axis=1)
        q_ref[:] = jnp.sum(emb * emb, axis=1, keepdims=True)

    xtb = xt_ref[:]
    emb = emb_ref[:]
    for e in range(EMBED):
        out_ref[:, e, :] = xtb * emb[:, e:e + 1]

    p = lax.dot_general(a_ref[:], xtb, (((0,), (0,)), ((), ())),
                        precision=hi, preferred_element_type=f32)
    sq = lax.dot_general(q_ref[:], xtb * xtb, (((0,), (0,)), ((), ())),
                         precision=hi, preferred_element_type=f32)
    inter = 0.5 * (p[1:2] * p[1:2] - sq)
    yfm_ref[:] = jnp.concatenate([p[0:1], inter], axis=0)


def kernel(x, w, V, field_index):
    batch = x.shape[0]
    xt = x.T
    w2 = w.reshape(NUM_FEATURES, 1)
    fi2 = field_index.reshape(NUM_FEATURES, 1)
    grid = batch // CB
    yfm_t, out_p = pl.pallas_call(
        _fm_body,
        grid=(grid,),
        in_specs=[
            pl.BlockSpec((NUM_FEATURES, CB), lambda i: (0, i)),
            pl.BlockSpec((NUM_FEATURES, 1), lambda i: (0, 0)),
            pl.BlockSpec((NUM_FIELDS, EMBED), lambda i: (0, 0)),
            pl.BlockSpec((NUM_FEATURES, 1), lambda i: (0, 0)),
        ],
        out_specs=[
            pl.BlockSpec((2, CB), lambda i: (0, i)),
            pl.BlockSpec((NUM_FEATURES, EMBED, CB), lambda i: (0, 0, i)),
        ],
        out_shape=[
            jax.ShapeDtypeStruct((2, batch), jnp.float32),
            jax.ShapeDtypeStruct((NUM_FEATURES, EMBED, batch), jnp.float32),
        ],
        scratch_shapes=[
            pltpu.VMEM((NUM_FEATURES, EMBED), jnp.float32),
            pltpu.VMEM((NUM_FEATURES, 2), jnp.float32),
            pltpu.VMEM((NUM_FEATURES, 1), jnp.float32),
        ],
        compiler_params=pltpu.CompilerParams(
            dimension_semantics=("arbitrary",)),
    )(xt, w2, V, fi2)
    return (yfm_t.T, jnp.transpose(out_p, (2, 0, 1)))
